# Initial kernel scaffold; baseline (speedup 1.0000x reference)
import jax
import jax.numpy as jnp
from jax.experimental import pallas as pl


def kernel(feature, label):
    raise NotImplementedError("write your pallas kernel here")


# speedup vs baseline: 1.2577x; 1.2577x over previous
import jax
import jax.numpy as jnp
from jax.experimental import pallas as pl
from jax.experimental.pallas import tpu as pltpu

B_TOTAL = 262144
K = 128
BLK = 2048
NCORE = 2
NSTEP = B_TOTAL // (NCORE * BLK)


def _accum_kernel(f_ref, w_ref, g0_ref, ga_ref, vec_ref):
    j = pl.program_id(1)

    @pl.when(j == 0)
    def _():
        g0_ref[...] = jnp.zeros_like(g0_ref)
        ga_ref[...] = jnp.zeros_like(ga_ref)
        vec_ref[...] = jnp.zeros_like(vec_ref)

    f = f_ref[...]
    w1 = w_ref[...]
    w0 = 1.0 - w1
    fw = f * w0

    g0 = jax.lax.dot_general(
        fw, f, (((0,), (0,)), ((), ())),
        preferred_element_type=jnp.float32,
        precision=jax.lax.Precision.HIGHEST)
    ga = jax.lax.dot_general(
        f, f, (((0,), (0,)), ((), ())),
        preferred_element_type=jnp.float32,
        precision=jax.lax.Precision.HIGHEST)
    g0_ref[...] += g0[None]
    ga_ref[...] += ga[None]

    sum0 = jnp.sum(fw, axis=0, keepdims=True)
    suma = jnp.sum(f, axis=0, keepdims=True)
    n0 = jnp.sum(w0, axis=0, keepdims=True)
    n0row = jnp.broadcast_to(n0, (1, K))
    pad = jnp.zeros((5, K), jnp.float32)
    vec_ref[...] += jnp.concatenate([sum0, suma, n0row, pad], axis=0)[None]


def _outer(a, b):
    return jax.lax.dot_general(
        a, b, (((0,), (0,)), ((), ())), preferred_element_type=jnp.float32)


def _finalize_kernel(g0_ref, ga_ref, vec_ref, o_ref):
    G0 = g0_ref[0] + g0_ref[1]
    GA = ga_ref[0] + ga_ref[1]
    v = vec_ref[0] + vec_ref[1]
    sum0 = v[0:1, :]
    suma = v[1:2, :]
    n0 = v[2:3, 0:1]
    n1 = float(B_TOTAL) - n0
    r0 = 1.0 / n0
    r1 = 1.0 / n1
    sum1 = suma - sum0
    mu1 = sum0 * r0
    mu2 = sum1 * r1

    S1 = G0 - _outer(sum0, sum0) * r0
    G1 = GA - G0
    S2 = G1 - _outer(mu1, sum1) - _outer(sum1, mu1) + n1 * _outer(mu1, mu1)

    ri = jax.lax.broadcasted_iota(jnp.int32, (K, 1), 0)
    ci = jax.lax.broadcasted_iota(jnp.int32, (1, K), 1)
    eye = (ri == ci).astype(jnp.float32)
    Sf1 = eye + S1 * r0
    Sf2 = eye + S2 * r1

    def gj_step(j, carry):
        M, Inv, ld = carry
        ej_row = (ci == j).astype(jnp.float32)
        ej_col = (ri == j).astype(jnp.float32)
        rowm = jnp.sum(M * ej_col, axis=0, keepdims=True)
        rowi = jnp.sum(Inv * ej_col, axis=0, keepdims=True)
        colm = jnp.sum(M * ej_row, axis=1, keepdims=True)
        p = jnp.sum(rowm * ej_row, axis=1, keepdims=True)
        rp = 1.0 / p
        cm = colm - ej_col
        M = M - cm * (rowm * rp)
        Inv = Inv - cm * (rowi * rp)
        ld = ld + (jnp.log2(p) - 1.0)
        return (M, Inv, ld)

    ld0 = jnp.zeros((1, 1), jnp.float32)
    _, inv2, ld2 = jax.lax.fori_loop(0, K, gj_step, (Sf2, eye, ld0))

    def det_step(j, carry):
        M, ld = carry
        ej_row = (ci == j).astype(jnp.float32)
        ej_col = (ri == j).astype(jnp.float32)
        rowm = jnp.sum(M * ej_col, axis=0, keepdims=True)
        colm = jnp.sum(M * ej_row, axis=1, keepdims=True)
        p = jnp.sum(rowm * ej_row, axis=1, keepdims=True)
        M = M - (colm - ej_col) * (rowm / p)
        ld = ld + (jnp.log2(p) - 1.0)
        return (M, ld)

    _, ld1 = jax.lax.fori_loop(0, K, det_step, (Sf1, ld0))

    d = mu1 - mu2
    quad = jnp.sum(inv2 * _outer(d, d), keepdims=True)[0:1, 0:1]
    trd = jnp.sum(inv2 * (Sf1 - Sf2), keepdims=True)[0:1, 0:1]
    o_ref[...] = 0.5 * ((ld2 - ld1) + quad + trd)


def kernel(feature, label):
    labf = label.astype(jnp.float32).reshape(B_TOTAL, 1)
    g0, ga, vec = pl.pallas_call(
        _accum_kernel,
        grid=(NCORE, NSTEP),
        in_specs=[
            pl.BlockSpec((BLK, K), lambda c, j: (c * NSTEP + j, 0)),
            pl.BlockSpec((BLK, 1), lambda c, j: (c * NSTEP + j, 0)),
        ],
        out_specs=[
            pl.BlockSpec((1, K, K), lambda c, j: (c, 0, 0)),
            pl.BlockSpec((1, K, K), lambda c, j: (c, 0, 0)),
            pl.BlockSpec((1, 8, K), lambda c, j: (c, 0, 0)),
        ],
        out_shape=[
            jax.ShapeDtypeStruct((NCORE, K, K), jnp.float32),
            jax.ShapeDtypeStruct((NCORE, K, K), jnp.float32),
            jax.ShapeDtypeStruct((NCORE, 8, K), jnp.float32),
        ],
        compiler_params=pltpu.CompilerParams(
            dimension_semantics=("parallel", "arbitrary"),
        ),
    )(feature, labf)
    out = pl.pallas_call(
        _finalize_kernel,
        out_shape=jax.ShapeDtypeStruct((1, 1), jnp.float32),
    )(g0, ga, vec)
    return out


# speedup vs baseline: 2.0290x; 1.6132x over previous
import jax
import jax.numpy as jnp
from jax.experimental import pallas as pl
from jax.experimental.pallas import tpu as pltpu

B_TOTAL = 262144
K = 128
BLK = 2048
NSTEP = B_TOTAL // BLK


def _outer(a, b):
    return jax.lax.dot_general(
        a, b, (((0,), (0,)), ((), ())), preferred_element_type=jnp.float32)


def _fused_kernel(f_ref, w_ref, o_ref, mh_ref, mw_ref, vec_ref):
    j = pl.program_id(0)

    @pl.when(j == 0)
    def _():
        mh_ref[...] = jnp.zeros_like(mh_ref)
        mw_ref[...] = jnp.zeros_like(mw_ref)
        vec_ref[...] = jnp.zeros_like(vec_ref)

    f = f_ref[...]
    w1 = w_ref[...]
    w0 = 1.0 - w1
    fw = f * w0

    h = f.astype(jnp.bfloat16)
    l = (f - h.astype(jnp.float32)).astype(jnp.bfloat16)
    hw = fw.astype(jnp.bfloat16)
    rhs = jnp.concatenate([h, l], axis=1)
    mh = jax.lax.dot_general(
        h, rhs, (((0,), (0,)), ((), ())),
        preferred_element_type=jnp.float32)
    mw = jax.lax.dot_general(
        hw, rhs, (((0,), (0,)), ((), ())),
        preferred_element_type=jnp.float32)
    mh_ref[...] += mh
    mw_ref[...] += mw

    sum0 = jnp.sum(fw, axis=0, keepdims=True)
    suma = jnp.sum(f, axis=0, keepdims=True)
    n0 = jnp.sum(w0, axis=0, keepdims=True)
    n0row = jnp.broadcast_to(n0, (1, K))
    pad = jnp.zeros((5, K), jnp.float32)
    vec_ref[...] += jnp.concatenate([sum0, suma, n0row, pad], axis=0)

    @pl.when(j == NSTEP - 1)
    def _():
        MH = mh_ref[...]
        MW = mw_ref[...]
        C = MH[:, K:]
        G0c = MW[:, K:]
        GA = MH[:, :K] + C + C.T
        G0 = MW[:, :K] + G0c + G0c.T
        v = vec_ref[...]
        sum0_ = v[0:1, :]
        suma_ = v[1:2, :]
        n0_ = v[2:3, 0:1]
        n1_ = float(B_TOTAL) - n0_
        r0 = 1.0 / n0_
        r1 = 1.0 / n1_
        sum1_ = suma_ - sum0_
        mu1 = sum0_ * r0
        mu2 = sum1_ * r1

        S1 = G0 - _outer(sum0_, sum0_) * r0
        G1 = GA - G0
        S2 = (G1 - _outer(mu1, sum1_) - _outer(sum1_, mu1)
              + n1_ * _outer(mu1, mu1))

        ri = jax.lax.broadcasted_iota(jnp.int32, (K, 1), 0)
        ci = jax.lax.broadcasted_iota(jnp.int32, (1, K), 1)
        eye = (ri == ci).astype(jnp.float32)
        Sf1 = eye + S1 * r0
        Sf2 = eye + S2 * r1

        def gj_step(i, carry):
            M, Inv, ld = carry
            ej_row = (ci == i).astype(jnp.float32)
            ej_col = (ri == i).astype(jnp.float32)
            rowm = jnp.sum(M * ej_col, axis=0, keepdims=True)
            rowi = jnp.sum(Inv * ej_col, axis=0, keepdims=True)
            colm = jnp.sum(M * ej_row, axis=1, keepdims=True)
            p = jnp.sum(rowm * ej_row, axis=1, keepdims=True)
            rp = 1.0 / p
            cm = colm - ej_col
            M = M - cm * (rowm * rp)
            Inv = Inv - cm * (rowi * rp)
            ld = ld + (jnp.log2(p) - 1.0)
            return (M, Inv, ld)

        ld0 = jnp.zeros((1, 1), jnp.float32)
        _, inv2, ld2 = jax.lax.fori_loop(0, K, gj_step, (Sf2, eye, ld0))

        def det_step(i, carry):
            M, ld = carry
            ej_row = (ci == i).astype(jnp.float32)
            ej_col = (ri == i).astype(jnp.float32)
            rowm = jnp.sum(M * ej_col, axis=0, keepdims=True)
            colm = jnp.sum(M * ej_row, axis=1, keepdims=True)
            p = jnp.sum(rowm * ej_row, axis=1, keepdims=True)
            M = M - (colm - ej_col) * (rowm / p)
            ld = ld + (jnp.log2(p) - 1.0)
            return (M, ld)

        _, ld1 = jax.lax.fori_loop(0, K, det_step, (Sf1, ld0))

        d = mu1 - mu2
        quad = jnp.sum(inv2 * _outer(d, d), keepdims=True)[0:1, 0:1]
        trd = jnp.sum(inv2 * (Sf1 - Sf2), keepdims=True)[0:1, 0:1]
        o_ref[...] = 0.5 * ((ld2 - ld1) + quad + trd)


def kernel(feature, label):
    labf = label.astype(jnp.float32).reshape(B_TOTAL, 1)
    out = pl.pallas_call(
        _fused_kernel,
        grid=(NSTEP,),
        in_specs=[
            pl.BlockSpec((BLK, K), lambda j: (j, 0)),
            pl.BlockSpec((BLK, 1), lambda j: (j, 0)),
        ],
        out_specs=pl.BlockSpec((1, 1), lambda j: (0, 0)),
        out_shape=jax.ShapeDtypeStruct((1, 1), jnp.float32),
        scratch_shapes=[
            pltpu.VMEM((K, 2 * K), jnp.float32),
            pltpu.VMEM((K, 2 * K), jnp.float32),
            pltpu.VMEM((8, K), jnp.float32),
        ],
        compiler_params=pltpu.CompilerParams(
            dimension_semantics=("arbitrary",),
        ),
    )(feature, labf)
    return out
